# baseline (speedup 1.0000x reference)
import functools

import jax
import jax.numpy as jnp
from jax import lax
from jax.experimental import pallas as pl
from jax.experimental.pallas import tpu as pltpu
from jax.experimental.pallas import tpu_sc as plsc

NUM_H = 256
NDIR = 26
DPAD = 32
NW = 32
BLK = 128
TPACK = 16
HSTRIDE = NUM_H + 1
HIST = DPAD * HSTRIDE


def _ceil_to(x: int, m: int) -> int:
    return (x + m - 1) // m * m


def _maxnorm2_body(c_ref, o_ref):
    i = pl.program_id(0)
    x = c_ref[...]
    m = jnp.max(jnp.sum(x * x, axis=0))

    @pl.when(i == 0)
    def _():
        o_ref[0, 0] = m

    @pl.when(i > 0)
    def _():
        o_ref[0, 0] = jnp.maximum(o_ref[0, 0], m)


def _dirs_body(mh2_ref, dT_ref, o_ref):
    eps = 1e-12
    mh = jnp.maximum(jnp.sqrt(mh2_ref[0, 0]), eps)
    dT = dT_ref[...]
    n = jnp.sqrt(jnp.sum(dT * dT, axis=0, keepdims=True))
    dn = dT / jnp.maximum(n, eps)
    s = (NUM_H - 1) / (2.0 * mh)
    o_ref[0:3, :] = dn
    o_ref[3:4, :] = jnp.full((1, DPAD), s, jnp.float32)
    o_ref[4:8, :] = jnp.zeros((4, DPAD), jnp.float32)


def _sc_bins(cx, cy, cz, vw, dnp, vb_blocks, vp2):
    mesh = plsc.VectorSubcoreMesh(core_axis_name="c", subcore_axis_name="s")

    col_t = pltpu.VMEM((BLK,), jnp.float32)
    slab_t = pltpu.VMEM((BLK, TPACK), jnp.int32)

    @functools.partial(
        pl.kernel, mesh=mesh,
        out_type=(jax.ShapeDtypeStruct((vp2, TPACK), jnp.int32),
                  jax.ShapeDtypeStruct((NW, HIST), jnp.float32)),
        compiler_params=pltpu.CompilerParams(needs_layout_passes=False,
                                             use_tc_tiling_on_sc=False),
        scratch_types=[
            pltpu.VMEM((8, DPAD), jnp.float32),
            [[col_t] * 4, [col_t] * 4],
            [slab_t, slab_t],
            pltpu.VMEM((HIST,), jnp.float32),
            [pltpu.SemaphoreType.DMA] * 2,
            [pltpu.SemaphoreType.DMA] * 2,
        ],
    )
    def k(cx_h, cy_h, cz_h, vw_h, dnp_h, tab_h, out_h,
          dn_v, cols, slabs, hist, semc, semw):
        wid = lax.axis_index("s") * 2 + lax.axis_index("c")
        per_w = vb_blocks * BLK

        pltpu.sync_copy(dnp_h, dn_v)
        dx = [dn_v[0, 0:16], dn_v[0, 16:32]]
        dy = [dn_v[1, 0:16], dn_v[1, 16:32]]
        dz = [dn_v[2, 0:16], dn_v[2, 16:32]]
        srow = dn_v[3, 0:16]
        s = srow[0]

        lanes = lax.iota(jnp.int32, 16)
        offs = [lanes * HSTRIDE, lanes * HSTRIDE + 16 * HSTRIDE]

        @plsc.parallel_loop(0, HIST // 16)
        def _(i):
            hist[pl.ds(i * 16, 16)] = jnp.zeros((16,), jnp.float32)

        col_hs = [cx_h, cy_h, cz_h, vw_h]

        def issue_cols(j, b):
            base = wid * per_w + j * BLK
            for h, v in zip(col_hs, cols[b]):
                pltpu.async_copy(h.at[pl.ds(base, BLK)], v, semc[b])

        def wait_cols(b):
            for h, v in zip(col_hs, cols[b]):
                pltpu.make_async_copy(h.at[pl.ds(0, BLK)], v, semc[b]).wait()

        def compute_block(b):
            xs, ys, zs, ws = cols[b]
            slab = slabs[b]

            @plsc.parallel_loop(0, BLK // 16)
            def _(g):
                x = xs[pl.ds(g * 16, 16)]
                y = ys[pl.ds(g * 16, 16)]
                z = zs[pl.ds(g * 16, 16)]
                bins = []
                for d in range(DPAD):
                    hf, dd = divmod(d, 16)
                    if d >= NDIR:
                        bins.append(None)
                        continue
                    h = x * dx[hf][dd] + y * dy[hf][dd] + z * dz[hf][dd]
                    u = h * s + (NUM_H - 1) * 0.5
                    t = u.astype(jnp.int32)
                    t = jnp.where(t.astype(jnp.float32) < u, t + 1, t)
                    bins.append(jnp.clip(t, 0, NUM_H - 1))
                words = []
                for kk in range(TPACK):
                    w = bins[kk]
                    if bins[kk + 16] is not None:
                        w = w | (bins[kk + 16] << 16)
                    words.append(w)
                rowi = lanes + g * 16
                for kk in range(TPACK):
                    plsc.store_scatter(
                        slab, [rowi, jnp.full((16,), kk, jnp.int32)],
                        words[kk])
                w16 = ws[pl.ds(g * 16, 16)]
                for j in range(16):
                    wv = jnp.full((16,), w16[j], dtype=jnp.float32)
                    v = slab[g * 16 + j, :]
                    lo = v & 0xFFFF
                    hi = lax.shift_right_logical(v, 16)
                    plsc.addupdate_scatter(hist, [lo + offs[0]], wv)
                    plsc.addupdate_scatter(hist, [hi + offs[1]], wv)

        def wait_slab(b):
            pltpu.make_async_copy(tab_h.at[pl.ds(0, BLK)], slabs[b],
                                  semw[b]).wait()

        issue_cols(0, 0)
        issue_cols(1, 1)

        def body(g, _):
            for b in (0, 1):
                j = 2 * g + b
                wait_cols(b)
                @pl.when(j >= 2)
                def _():
                    wait_slab(b)
                compute_block(b)
                base = wid * per_w + j * BLK
                pltpu.async_copy(slabs[b], tab_h.at[pl.ds(base, BLK)], semw[b])
                issue_cols(j + 2, b)
            return 0

        lax.fori_loop(0, vb_blocks // 2, body, 0)
        wait_slab(0)
        wait_slab(1)
        wait_cols(0)
        wait_cols(1)

        pltpu.sync_copy(hist, out_h.at[wid])

    return k(cx, cy, cz, vw, dnp)


def _sc_simplices(table, ea, eb, ew, ta, tb, tc, tw, eb_blocks, tb_blocks):
    mesh = plsc.VectorSubcoreMesh(core_axis_name="c", subcore_axis_name="s")

    idx_t = pltpu.VMEM((BLK,), jnp.int32)
    row_t = pltpu.VMEM((BLK, TPACK), jnp.int32)
    w_t = pltpu.VMEM((BLK,), jnp.float32)

    @functools.partial(
        pl.kernel, mesh=mesh,
        out_type=jax.ShapeDtypeStruct((NW, 2 * HIST), jnp.float32),
        compiler_params=pltpu.CompilerParams(needs_layout_passes=False,
                                             use_tc_tiling_on_sc=False),
        scratch_types=[
            [idx_t] * 3, [idx_t] * 3,
            [row_t] * 3, [row_t] * 3,
            [w_t, w_t],
            pltpu.VMEM((2 * HIST,), jnp.float32),
            [pltpu.SemaphoreType.DMA] * 2,
            [pltpu.SemaphoreType.DMA] * 2,
        ],
    )
    def k(table_h, ea_h, eb_h, ta_h, tb_h, tc_h, ew_h, tw_h, out_h,
          id0, id1, rows0, rows1, wbs, hist, semi, semg):
        wid = lax.axis_index("s") * 2 + lax.axis_index("c")
        ids = [id0, id1]
        rows = [rows0, rows1]

        lanes = lax.iota(jnp.int32, 16)
        offs = [lanes * HSTRIDE, lanes * HSTRIDE + 16 * HSTRIDE]

        @plsc.parallel_loop(0, 2 * HIST // 16)
        def _(i):
            hist[pl.ds(i * 16, 16)] = jnp.zeros((16,), jnp.float32)

        def inner(row_refs, wb, sign):
            @plsc.parallel_loop(0, BLK // 16, unroll=2)
            def _(g):
                w16 = wb[pl.ds(g * 16, 16)]
                if sign < 0:
                    w16 = -w16
                for j in range(16):
                    wv = jnp.full((16,), w16[j], dtype=jnp.float32)
                    s = g * 16 + j
                    h2 = (j & 1) * HIST
                    vs = [r[s, :] for r in row_refs]
                    lo = vs[0] & 0xFFFF
                    hi = lax.shift_right_logical(vs[0], 16)
                    for v in vs[1:]:
                        lo = jnp.maximum(lo, v & 0xFFFF)
                        hi = jnp.maximum(hi, lax.shift_right_logical(v, 16))
                    plsc.addupdate_scatter(hist, [lo + offs[0] + h2], wv)
                    plsc.addupdate_scatter(hist, [hi + offs[1] + h2], wv)

        def simplex_phase(nblk, id_hs, w_h, sign):
            per_w = nblk * BLK
            nh = len(id_hs)

            def issue_idx(j, b):
                base = wid * per_w + j * BLK
                for h, v in zip(id_hs, ids[b]):
                    pltpu.async_copy(h.at[pl.ds(base, BLK)], v, semi[b])
                pltpu.async_copy(w_h.at[pl.ds(base, BLK)], wbs[b], semi[b])

            def wait_idx(b):
                for h, v in zip(id_hs, ids[b]):
                    pltpu.make_async_copy(h.at[pl.ds(0, BLK)], v,
                                          semi[b]).wait()
                pltpu.make_async_copy(w_h.at[pl.ds(0, BLK)], wbs[b],
                                      semi[b]).wait()

            def issue_gather(b):
                for iv, rv in zip(ids[b][:nh], rows[b]):
                    pltpu.async_copy(table_h.at[iv], rv, semg[b])

            def wait_gather(b):
                for iv, rv in zip(ids[b][:nh], rows[b]):
                    pltpu.make_async_copy(table_h.at[iv], rv, semg[b]).wait()

            issue_idx(0, 0)
            wait_idx(0)
            issue_gather(0)
            issue_idx(1, 1)

            def body(g, _):
                for b in (0, 1):
                    j = 2 * g + b
                    nb = 1 - b
                    wait_idx(nb)
                    issue_gather(nb)
                    wait_gather(b)
                    inner(rows[b][:nh], wbs[b], sign)
                    issue_idx(j + 2, b)
                return 0

            lax.fori_loop(0, nblk // 2, body, 0)
            wait_gather(0)
            wait_idx(1)

        simplex_phase(eb_blocks, [ea_h, eb_h], ew_h, -1)
        simplex_phase(tb_blocks, [ta_h, tb_h, tc_h], tw_h, 1)

        pltpu.sync_copy(hist, out_h.at[wid])

    return k(table, ea, eb, ta, tb, tc, ew, tw)


def _finish_body(p_ref, o_ref):
    s = jnp.sum(p_ref[...], axis=0)
    r = lax.broadcasted_iota(jnp.int32, (NUM_H, NUM_H), 0)
    c = lax.broadcasted_iota(jnp.int32, (NUM_H, NUM_H), 1)
    tri = (r <= c).astype(jnp.float32)
    cum = lax.dot_general(s, tri, (((1,), (0,)), ((), ())),
                          precision=lax.Precision.HIGHEST,
                          preferred_element_type=jnp.float32)
    o_ref[...] = cum[:26, :]


def kernel(v_coords, v_weights, simp1_verts, simp1_weights, simp2_verts,
           simp2_weights, dirs):
    nv = v_coords.shape[0]
    ne = simp1_verts.shape[0]
    nt = simp2_verts.shape[0]

    vb_blocks = _ceil_to(_ceil_to(nv, NW * BLK) // (NW * BLK), 2)
    eb_blocks = _ceil_to(_ceil_to(ne, NW * BLK) // (NW * BLK), 2)
    tb_blocks = _ceil_to(_ceil_to(nt, NW * BLK) // (NW * BLK), 2)
    vp = vb_blocks * NW * BLK
    ep = eb_blocks * NW * BLK
    tp = tb_blocks * NW * BLK
    vp2 = vp + 2048

    coordsT = jnp.pad(v_coords.T, ((0, 0), (0, vp2 - nv)))
    cx = coordsT[0]
    cy = coordsT[1]
    cz = coordsT[2]
    vw = jnp.pad(v_weights, (0, vp2 - nv))
    ea = jnp.pad(simp1_verts[:, 0], (0, ep + 2 * BLK - ne))
    eb = jnp.pad(simp1_verts[:, 1], (0, ep + 2 * BLK - ne))
    ew = jnp.pad(simp1_weights, (0, ep + 2 * BLK - ne))
    ta = jnp.pad(simp2_verts[:, 0], (0, tp + 2 * BLK - nt))
    tb = jnp.pad(simp2_verts[:, 1], (0, tp + 2 * BLK - nt))
    tc = jnp.pad(simp2_verts[:, 2], (0, tp + 2 * BLK - nt))
    tw = jnp.pad(simp2_weights, (0, tp + 2 * BLK - nt))
    dirsT = jnp.pad(dirs.T, ((0, 0), (0, DPAD - dirs.shape[0])))

    ab = 8192
    mh2 = pl.pallas_call(
        _maxnorm2_body,
        grid=(vp2 // ab,),
        in_specs=[pl.BlockSpec((3, ab), lambda i: (0, i))],
        out_specs=pl.BlockSpec(memory_space=pltpu.SMEM),
        out_shape=jax.ShapeDtypeStruct((1, 1), jnp.float32),
    )(coordsT)

    dnp = pl.pallas_call(
        _dirs_body,
        in_specs=[
            pl.BlockSpec(memory_space=pltpu.SMEM),
            pl.BlockSpec((3, DPAD), lambda: (0, 0)),
        ],
        out_specs=pl.BlockSpec((8, DPAD), lambda: (0, 0)),
        out_shape=jax.ShapeDtypeStruct((8, DPAD), jnp.float32),
    )(mh2, dirsT)

    table, pv = _sc_bins(cx, cy, cz, vw, dnp, vb_blocks, vp2)
    pe = _sc_simplices(table, ea, eb, ew, ta, tb, tc, tw,
                       eb_blocks, tb_blocks)

    partials = jnp.concatenate([pv, pe.reshape(2 * NW, HIST)], axis=0)
    out = pl.pallas_call(
        _finish_body,
        in_specs=[pl.BlockSpec((3 * NW, DPAD, NUM_H), lambda: (0, 0, 0))],
        out_specs=pl.BlockSpec((26, NUM_H), lambda: (0, 0)),
        out_shape=jax.ShapeDtypeStruct((26, NUM_H), jnp.float32),
    )(partials.reshape(3 * NW, DPAD, HSTRIDE)[:, :, :NUM_H])
    return out


# speedup vs baseline: 1.0101x; 1.0101x over previous
import functools

import jax
import jax.numpy as jnp
from jax import lax
from jax.experimental import pallas as pl
from jax.experimental.pallas import tpu as pltpu
from jax.experimental.pallas import tpu_sc as plsc

NUM_H = 256
NDIR = 26
DPAD = 32
NW = 32
BLK = 128
TPACK = 16
HSTRIDE = NUM_H + 1
HIST = DPAD * HSTRIDE


def _ceil_to(x: int, m: int) -> int:
    return (x + m - 1) // m * m


def _maxnorm2_body(c_ref, o_ref):
    i = pl.program_id(0)
    x = c_ref[...]
    m = jnp.max(jnp.sum(x * x, axis=0))

    @pl.when(i == 0)
    def _():
        o_ref[0, 0] = m

    @pl.when(i > 0)
    def _():
        o_ref[0, 0] = jnp.maximum(o_ref[0, 0], m)


def _dirs_body(mh2_ref, dT_ref, o_ref):
    eps = 1e-12
    mh = jnp.maximum(jnp.sqrt(mh2_ref[0, 0]), eps)
    dT = dT_ref[...]
    n = jnp.sqrt(jnp.sum(dT * dT, axis=0, keepdims=True))
    dn = dT / jnp.maximum(n, eps)
    s = (NUM_H - 1) / (2.0 * mh)
    o_ref[0:3, :] = dn
    o_ref[3:4, :] = jnp.full((1, DPAD), s, jnp.float32)
    o_ref[4:8, :] = jnp.zeros((4, DPAD), jnp.float32)


def _sc_bins(cx, cy, cz, vw, dnp, vb_blocks, vp2):
    mesh = plsc.VectorSubcoreMesh(core_axis_name="c", subcore_axis_name="s")

    col_t = pltpu.VMEM((BLK,), jnp.float32)
    slab_t = pltpu.VMEM((BLK, TPACK), jnp.int32)

    @functools.partial(
        pl.kernel, mesh=mesh,
        out_type=(jax.ShapeDtypeStruct((vp2, TPACK), jnp.int32),
                  jax.ShapeDtypeStruct((NW, HIST), jnp.float32)),
        compiler_params=pltpu.CompilerParams(needs_layout_passes=False,
                                             use_tc_tiling_on_sc=False),
        scratch_types=[
            pltpu.VMEM((8, DPAD), jnp.float32),
            [[col_t] * 4, [col_t] * 4],
            [slab_t, slab_t],
            pltpu.VMEM((HIST,), jnp.float32),
            [pltpu.SemaphoreType.DMA] * 2,
            [pltpu.SemaphoreType.DMA] * 2,
        ],
    )
    def k(cx_h, cy_h, cz_h, vw_h, dnp_h, tab_h, out_h,
          dn_v, cols, slabs, hist, semc, semw):
        wid = lax.axis_index("s") * 2 + lax.axis_index("c")
        per_w = vb_blocks * BLK

        pltpu.sync_copy(dnp_h, dn_v)
        dx = [dn_v[0, 0:16], dn_v[0, 16:32]]
        dy = [dn_v[1, 0:16], dn_v[1, 16:32]]
        dz = [dn_v[2, 0:16], dn_v[2, 16:32]]
        srow = dn_v[3, 0:16]
        s = srow[0]

        lanes = lax.iota(jnp.int32, 16)
        offs = [lanes * HSTRIDE, lanes * HSTRIDE + 16 * HSTRIDE]

        @plsc.parallel_loop(0, HIST // 16)
        def _(i):
            hist[pl.ds(i * 16, 16)] = jnp.zeros((16,), jnp.float32)

        col_hs = [cx_h, cy_h, cz_h, vw_h]

        def issue_cols(j, b):
            base = wid * per_w + j * BLK
            for h, v in zip(col_hs, cols[b]):
                pltpu.async_copy(h.at[pl.ds(base, BLK)], v, semc[b])

        def wait_cols(b):
            for h, v in zip(col_hs, cols[b]):
                pltpu.make_async_copy(h.at[pl.ds(0, BLK)], v, semc[b]).wait()

        def compute_block(b):
            xs, ys, zs, ws = cols[b]
            slab = slabs[b]

            @plsc.parallel_loop(0, BLK // 16)
            def _(g):
                x = xs[pl.ds(g * 16, 16)]
                y = ys[pl.ds(g * 16, 16)]
                z = zs[pl.ds(g * 16, 16)]
                bins = []
                for d in range(DPAD):
                    hf, dd = divmod(d, 16)
                    if d >= NDIR:
                        bins.append(None)
                        continue
                    h = x * dx[hf][dd] + y * dy[hf][dd] + z * dz[hf][dd]
                    u = h * s + (NUM_H - 1) * 0.5
                    t = u.astype(jnp.int32)
                    t = jnp.where(t.astype(jnp.float32) < u, t + 1, t)
                    bins.append(jnp.clip(t, 0, NUM_H - 1))
                words = []
                for kk in range(TPACK):
                    w = bins[kk]
                    if bins[kk + 16] is not None:
                        w = w | (bins[kk + 16] << 16)
                    words.append(w)
                rowi = lanes + g * 16
                for kk in range(TPACK):
                    plsc.store_scatter(
                        slab, [rowi, jnp.full((16,), kk, jnp.int32)],
                        words[kk])
                w16 = ws[pl.ds(g * 16, 16)]
                for j in range(16):
                    wv = jnp.full((16,), w16[j], dtype=jnp.float32)
                    v = slab[g * 16 + j, :]
                    lo = v & 0xFFFF
                    hi = lax.shift_right_logical(v, 16)
                    plsc.addupdate_scatter(hist, [lo + offs[0]], wv)
                    plsc.addupdate_scatter(hist, [hi + offs[1]], wv)

        def wait_slab(b):
            pltpu.make_async_copy(tab_h.at[pl.ds(0, BLK)], slabs[b],
                                  semw[b]).wait()

        issue_cols(0, 0)
        issue_cols(1, 1)

        def body(g, _):
            for b in (0, 1):
                j = 2 * g + b
                wait_cols(b)
                @pl.when(j >= 2)
                def _():
                    wait_slab(b)
                compute_block(b)
                base = wid * per_w + j * BLK
                pltpu.async_copy(slabs[b], tab_h.at[pl.ds(base, BLK)], semw[b])
                issue_cols(j + 2, b)
            return 0

        lax.fori_loop(0, vb_blocks // 2, body, 0)
        wait_slab(0)
        wait_slab(1)
        wait_cols(0)
        wait_cols(1)

        pltpu.sync_copy(hist, out_h.at[wid])

    return k(cx, cy, cz, vw, dnp)


def _sc_simplices(table, ea, eb, ew, ta, tb, tc, tw, eb_blocks, tb_blocks):
    mesh = plsc.VectorSubcoreMesh(core_axis_name="c", subcore_axis_name="s")

    idx_t = pltpu.VMEM((BLK,), jnp.int32)
    row_t = pltpu.VMEM((BLK, TPACK), jnp.int32)
    w_t = pltpu.VMEM((BLK,), jnp.float32)

    @functools.partial(
        pl.kernel, mesh=mesh,
        out_type=jax.ShapeDtypeStruct((NW, HIST), jnp.float32),
        compiler_params=pltpu.CompilerParams(needs_layout_passes=False,
                                             use_tc_tiling_on_sc=False),
        scratch_types=[
            [idx_t] * 3, [idx_t] * 3,
            [row_t] * 3, [row_t] * 3,
            [w_t, w_t],
            pltpu.VMEM((HIST,), jnp.float32),
            [pltpu.SemaphoreType.DMA] * 2,
            [pltpu.SemaphoreType.DMA] * 2,
        ],
    )
    def k(table_h, ea_h, eb_h, ta_h, tb_h, tc_h, ew_h, tw_h, out_h,
          id0, id1, rows0, rows1, wbs, hist, semi, semg):
        wid = lax.axis_index("s") * 2 + lax.axis_index("c")
        ids = [id0, id1]
        rows = [rows0, rows1]

        lanes = lax.iota(jnp.int32, 16)
        offs = [lanes * HSTRIDE, lanes * HSTRIDE + 16 * HSTRIDE]

        @plsc.parallel_loop(0, HIST // 16)
        def _(i):
            hist[pl.ds(i * 16, 16)] = jnp.zeros((16,), jnp.float32)

        def inner(row_refs, wb, sign):
            @plsc.parallel_loop(0, BLK // 16, unroll=2)
            def _(g):
                w16 = wb[pl.ds(g * 16, 16)]
                if sign < 0:
                    w16 = -w16
                for j in range(16):
                    wv = jnp.full((16,), w16[j], dtype=jnp.float32)
                    s = g * 16 + j
                    vs = [r[s, :] for r in row_refs]
                    lo = vs[0] & 0xFFFF
                    hi = lax.shift_right_logical(vs[0], 16)
                    for v in vs[1:]:
                        lo = jnp.maximum(lo, v & 0xFFFF)
                        hi = jnp.maximum(hi, lax.shift_right_logical(v, 16))
                    plsc.addupdate_scatter(hist, [lo + offs[0]], wv)
                    plsc.addupdate_scatter(hist, [hi + offs[1]], wv)

        def simplex_phase(nblk, id_hs, w_h, sign):
            per_w = nblk * BLK
            nh = len(id_hs)

            def issue_idx(j, b):
                base = wid * per_w + j * BLK
                for h, v in zip(id_hs, ids[b]):
                    pltpu.async_copy(h.at[pl.ds(base, BLK)], v, semi[b])
                pltpu.async_copy(w_h.at[pl.ds(base, BLK)], wbs[b], semi[b])

            def wait_idx(b):
                for h, v in zip(id_hs, ids[b]):
                    pltpu.make_async_copy(h.at[pl.ds(0, BLK)], v,
                                          semi[b]).wait()
                pltpu.make_async_copy(w_h.at[pl.ds(0, BLK)], wbs[b],
                                      semi[b]).wait()

            def issue_gather(b):
                for iv, rv in zip(ids[b][:nh], rows[b]):
                    pltpu.async_copy(table_h.at[iv], rv, semg[b])

            def wait_gather(b):
                for iv, rv in zip(ids[b][:nh], rows[b]):
                    pltpu.make_async_copy(table_h.at[iv], rv, semg[b]).wait()

            issue_idx(0, 0)
            wait_idx(0)
            issue_gather(0)
            issue_idx(1, 1)

            def body(g, _):
                for b in (0, 1):
                    j = 2 * g + b
                    nb = 1 - b
                    wait_idx(nb)
                    issue_gather(nb)
                    wait_gather(b)
                    inner(rows[b][:nh], wbs[b], sign)
                    issue_idx(j + 2, b)
                return 0

            lax.fori_loop(0, nblk // 2, body, 0)
            wait_gather(0)
            wait_idx(1)

        simplex_phase(eb_blocks, [ea_h, eb_h], ew_h, -1)
        simplex_phase(tb_blocks, [ta_h, tb_h, tc_h], tw_h, 1)

        pltpu.sync_copy(hist, out_h.at[wid])

    return k(table, ea, eb, ta, tb, tc, ew, tw)


def _finish_body(p_ref, o_ref):
    s = jnp.sum(p_ref[...], axis=0)
    r = lax.broadcasted_iota(jnp.int32, (NUM_H, NUM_H), 0)
    c = lax.broadcasted_iota(jnp.int32, (NUM_H, NUM_H), 1)
    tri = (r <= c).astype(jnp.float32)
    cum = lax.dot_general(s, tri, (((1,), (0,)), ((), ())),
                          precision=lax.Precision.HIGHEST,
                          preferred_element_type=jnp.float32)
    o_ref[...] = cum[:26, :]


def kernel(v_coords, v_weights, simp1_verts, simp1_weights, simp2_verts,
           simp2_weights, dirs):
    nv = v_coords.shape[0]
    ne = simp1_verts.shape[0]
    nt = simp2_verts.shape[0]

    vb_blocks = _ceil_to(_ceil_to(nv, NW * BLK) // (NW * BLK), 2)
    eb_blocks = _ceil_to(_ceil_to(ne, NW * BLK) // (NW * BLK), 2)
    tb_blocks = _ceil_to(_ceil_to(nt, NW * BLK) // (NW * BLK), 2)
    vp = vb_blocks * NW * BLK
    ep = eb_blocks * NW * BLK
    tp = tb_blocks * NW * BLK
    vp2 = vp + 2048

    coordsT = jnp.pad(v_coords.T, ((0, 0), (0, vp2 - nv)))
    cx = coordsT[0]
    cy = coordsT[1]
    cz = coordsT[2]
    vw = jnp.pad(v_weights, (0, vp2 - nv))
    ea = jnp.pad(simp1_verts[:, 0], (0, ep + 2 * BLK - ne))
    eb = jnp.pad(simp1_verts[:, 1], (0, ep + 2 * BLK - ne))
    ew = jnp.pad(simp1_weights, (0, ep + 2 * BLK - ne))
    ta = jnp.pad(simp2_verts[:, 0], (0, tp + 2 * BLK - nt))
    tb = jnp.pad(simp2_verts[:, 1], (0, tp + 2 * BLK - nt))
    tc = jnp.pad(simp2_verts[:, 2], (0, tp + 2 * BLK - nt))
    tw = jnp.pad(simp2_weights, (0, tp + 2 * BLK - nt))
    dirsT = jnp.pad(dirs.T, ((0, 0), (0, DPAD - dirs.shape[0])))

    ab = 8192
    mh2 = pl.pallas_call(
        _maxnorm2_body,
        grid=(vp2 // ab,),
        in_specs=[pl.BlockSpec((3, ab), lambda i: (0, i))],
        out_specs=pl.BlockSpec(memory_space=pltpu.SMEM),
        out_shape=jax.ShapeDtypeStruct((1, 1), jnp.float32),
    )(coordsT)

    dnp = pl.pallas_call(
        _dirs_body,
        in_specs=[
            pl.BlockSpec(memory_space=pltpu.SMEM),
            pl.BlockSpec((3, DPAD), lambda: (0, 0)),
        ],
        out_specs=pl.BlockSpec((8, DPAD), lambda: (0, 0)),
        out_shape=jax.ShapeDtypeStruct((8, DPAD), jnp.float32),
    )(mh2, dirsT)

    table, pv = _sc_bins(cx, cy, cz, vw, dnp, vb_blocks, vp2)
    pe = _sc_simplices(table, ea, eb, ew, ta, tb, tc, tw,
                       eb_blocks, tb_blocks)

    partials = jnp.concatenate([pv, pe], axis=0)
    out = pl.pallas_call(
        _finish_body,
        in_specs=[pl.BlockSpec((2 * NW, DPAD, NUM_H), lambda: (0, 0, 0))],
        out_specs=pl.BlockSpec((26, NUM_H), lambda: (0, 0)),
        out_shape=jax.ShapeDtypeStruct((26, NUM_H), jnp.float32),
    )(partials.reshape(2 * NW, DPAD, HSTRIDE)[:, :, :NUM_H])
    return out
